# baseline (device time: 96117 ns/iter reference)
import jax
import jax.numpy as jnp
from jax import lax
from jax.experimental import pallas as pl
from jax.experimental.pallas import tpu as pltpu

N_DEV = 16
B, SQ, D_MODEL = 2, 256, 512
HL, DH = 4, 64
BLK = 64
NBLK = SQ // BLK
ROWS = B * SQ
CHUNK = ROWS // N_DEV
HOPS = N_DEV - 1


def kernel(x, Wq, K_ext, V_ext, Wo):
    p = lax.axis_index("i")
    K_loc = jnp.transpose(
        lax.dynamic_slice_in_dim(K_ext, p * HL, HL, axis=2), (0, 2, 1, 3)
    ).astype(jnp.bfloat16)
    V_loc = jnp.transpose(
        lax.dynamic_slice_in_dim(V_ext, p * HL, HL, axis=2), (0, 2, 1, 3)
    ).astype(jnp.bfloat16)
    xb = x.astype(jnp.bfloat16)
    wq = Wq.astype(jnp.bfloat16)
    wo = Wo.astype(jnp.bfloat16)

    def body(x_ref, wq_ref, k_ref, v_ref, wo_ref, out_ref,
             acc_ref, comm_ref, rs_send, rs_recv, ag_send, ag_recv):
        my = lax.axis_index("i")
        right = lax.rem(my + 1, N_DEV)
        left = lax.rem(my + N_DEV - 1, N_DEV)

        for b in range(B):
            q_all = lax.dot(x_ref[b], wq_ref[...],
                            preferred_element_type=jnp.float32)
            ctx_heads = []
            for h in range(HL):
                qh = q_all[:, h * DH:(h + 1) * DH]
                kh = k_ref[b, h]
                vh = v_ref[b, h]
                blocks = []
                for qb in range(NBLK):
                    qblk = qh[qb * BLK:(qb + 1) * BLK, :].astype(jnp.bfloat16)
                    kblk = kh[qb * BLK:(qb + 1) * BLK, :]
                    s = lax.dot_general(
                        qblk, kblk, (((1,), (1,)), ((), ())),
                        preferred_element_type=jnp.float32) * 0.125
                    s = s - jnp.max(s, axis=1, keepdims=True)
                    e = jnp.exp(s)
                    w = e / jnp.sum(e, axis=1, keepdims=True)
                    vblk = vh[qb * BLK:(qb + 1) * BLK, :]
                    blocks.append(lax.dot(w.astype(jnp.bfloat16), vblk,
                                          preferred_element_type=jnp.float32))
                ctx_heads.append(jnp.concatenate(blocks, axis=0))
            ctx = jnp.concatenate(ctx_heads, axis=1)
            acc_ref[pl.ds(b * SQ, SQ), :] = lax.dot(
                ctx.astype(jnp.bfloat16), wo_ref[...],
                preferred_element_type=jnp.float32)

        barrier_sem = pltpu.get_barrier_semaphore()
        for nbr in (left, right):
            pl.semaphore_signal(barrier_sem, inc=1, device_id=(nbr,),
                                device_id_type=pl.DeviceIdType.MESH)
        pl.semaphore_wait(barrier_sem, 2)

        for s in range(HOPS):
            c_send = lax.rem(my + (N_DEV - s), N_DEV)
            c_recv = lax.rem(my + (N_DEV - 1 - s), N_DEV)
            rdma = pltpu.make_async_remote_copy(
                src_ref=acc_ref.at[pl.ds(c_send * CHUNK, CHUNK), :],
                dst_ref=comm_ref.at[s],
                send_sem=rs_send.at[s],
                recv_sem=rs_recv.at[s],
                device_id=(right,),
                device_id_type=pl.DeviceIdType.MESH,
            )
            rdma.start()
            rdma.wait()
            acc_ref[pl.ds(c_recv * CHUNK, CHUNK), :] = (
                acc_ref[pl.ds(c_recv * CHUNK, CHUNK), :] + comm_ref[s])

        for s in range(HOPS):
            c_send = lax.rem(my + (N_DEV + 1 - s), N_DEV)
            rdma = pltpu.make_async_remote_copy(
                src_ref=acc_ref.at[pl.ds(c_send * CHUNK, CHUNK), :],
                dst_ref=acc_ref.at[pl.ds(c_send * CHUNK, CHUNK), :],
                send_sem=ag_send.at[s],
                recv_sem=ag_recv.at[s],
                device_id=(right,),
                device_id_type=pl.DeviceIdType.MESH,
            )
            rdma.start()
            rdma.wait()

        for b in range(B):
            out_ref[b] = acc_ref[pl.ds(b * SQ, SQ), :]

    return pl.pallas_call(
        body,
        out_shape=jax.ShapeDtypeStruct((B, SQ, D_MODEL), jnp.float32),
        in_specs=[pl.BlockSpec(memory_space=pltpu.VMEM)] * 5,
        out_specs=pl.BlockSpec(memory_space=pltpu.VMEM),
        scratch_shapes=[
            pltpu.VMEM((ROWS, D_MODEL), jnp.float32),
            pltpu.VMEM((HOPS, CHUNK, D_MODEL), jnp.float32),
            pltpu.SemaphoreType.DMA((HOPS,)),
            pltpu.SemaphoreType.DMA((HOPS,)),
            pltpu.SemaphoreType.DMA((HOPS,)),
            pltpu.SemaphoreType.DMA((HOPS,)),
        ],
        compiler_params=pltpu.CompilerParams(collective_id=0),
    )(xb, wq, K_loc, V_loc, wo)


# device time: 36141 ns/iter; 2.6595x vs baseline; 2.6595x over previous
import jax
import jax.numpy as jnp
from jax import lax
from jax.experimental import pallas as pl
from jax.experimental.pallas import tpu as pltpu

N_DEV = 16
B, SQ, D_MODEL = 2, 256, 512
HL, DH = 4, 64
BLK = 64
NBLK = SQ // BLK
ROWS = B * SQ
CHUNK = ROWS // N_DEV


def kernel(x, Wq, K_ext, V_ext, Wo):
    p = lax.axis_index("i")
    K_loc = jnp.transpose(
        lax.dynamic_slice_in_dim(K_ext, p * HL, HL, axis=2), (0, 2, 1, 3)
    ).astype(jnp.bfloat16)
    V_loc = jnp.transpose(
        lax.dynamic_slice_in_dim(V_ext, p * HL, HL, axis=2), (0, 2, 1, 3)
    ).astype(jnp.bfloat16)
    xb = x.astype(jnp.bfloat16)
    wq = Wq.astype(jnp.bfloat16)
    wo = Wo.astype(jnp.bfloat16)

    def body(x_ref, wq_ref, k_ref, v_ref, wo_ref, out_ref,
             acc_ref, sbuf_ref, comm_ref, agbuf_ref,
             rs_send, rs_recv, ag_send, ag_recv):
        my = lax.axis_index("i")

        for b in range(B):
            q_all = lax.dot(x_ref[b], wq_ref[...],
                            preferred_element_type=jnp.float32)
            ctx_heads = []
            for h in range(HL):
                qh = q_all[:, h * DH:(h + 1) * DH]
                kh = k_ref[b, h]
                vh = v_ref[b, h]
                blocks = []
                for qb in range(NBLK):
                    qblk = qh[qb * BLK:(qb + 1) * BLK, :].astype(jnp.bfloat16)
                    kblk = kh[qb * BLK:(qb + 1) * BLK, :]
                    s = lax.dot_general(
                        qblk, kblk, (((1,), (1,)), ((), ())),
                        preferred_element_type=jnp.float32) * 0.125
                    s = s - jnp.max(s, axis=1, keepdims=True)
                    e = jnp.exp(s)
                    w = e / jnp.sum(e, axis=1, keepdims=True)
                    vblk = vh[qb * BLK:(qb + 1) * BLK, :]
                    blocks.append(lax.dot(w.astype(jnp.bfloat16), vblk,
                                          preferred_element_type=jnp.float32))
                ctx_heads.append(jnp.concatenate(blocks, axis=0))
            ctx = jnp.concatenate(ctx_heads, axis=1)
            acc_ref[pl.ds(b * SQ, SQ), :] = lax.dot(
                ctx.astype(jnp.bfloat16), wo_ref[...],
                preferred_element_type=jnp.float32)

        sbuf_ref[...] = acc_ref[...].astype(jnp.bfloat16)

        barrier_sem = pltpu.get_barrier_semaphore()
        for d in range(1, N_DEV):
            q = lax.rem(my + d, N_DEV)
            pl.semaphore_signal(barrier_sem, inc=1, device_id=(q,),
                                device_id_type=pl.DeviceIdType.MESH)
        pl.semaphore_wait(barrier_sem, N_DEV - 1)

        rs_descs = []
        for d in range(1, N_DEV):
            q = lax.rem(my + d, N_DEV)
            desc = pltpu.make_async_remote_copy(
                src_ref=sbuf_ref.at[pl.ds(q * CHUNK, CHUNK), :],
                dst_ref=comm_ref.at[my],
                send_sem=rs_send.at[d],
                recv_sem=rs_recv.at[my],
                device_id=(q,),
                device_id_type=pl.DeviceIdType.MESH,
            )
            desc.start()
            rs_descs.append(desc)

        comm_ref[my] = jnp.zeros((CHUNK, D_MODEL), jnp.bfloat16)
        for d in range(1, N_DEV):
            s = lax.rem(my + d, N_DEV)
            recv = pltpu.make_async_remote_copy(
                src_ref=sbuf_ref.at[pl.ds(0, CHUNK), :],
                dst_ref=comm_ref.at[s],
                send_sem=rs_send.at[0],
                recv_sem=rs_recv.at[s],
                device_id=(my,),
                device_id_type=pl.DeviceIdType.MESH,
            )
            recv.wait_recv()

        total = (acc_ref[pl.ds(my * CHUNK, CHUNK), :]
                 + jnp.sum(comm_ref[...].astype(jnp.float32), axis=0))
        agbuf_ref[pl.ds(my * CHUNK, CHUNK), :] = total.astype(jnp.bfloat16)

        ag_descs = []
        for d in range(1, N_DEV):
            q = lax.rem(my + d, N_DEV)
            desc = pltpu.make_async_remote_copy(
                src_ref=agbuf_ref.at[pl.ds(my * CHUNK, CHUNK), :],
                dst_ref=agbuf_ref.at[pl.ds(my * CHUNK, CHUNK), :],
                send_sem=ag_send.at[d],
                recv_sem=ag_recv.at[my],
                device_id=(q,),
                device_id_type=pl.DeviceIdType.MESH,
            )
            desc.start()
            ag_descs.append(desc)

        for d in range(1, N_DEV):
            s = lax.rem(my + d, N_DEV)
            recv = pltpu.make_async_remote_copy(
                src_ref=sbuf_ref.at[pl.ds(0, CHUNK), :],
                dst_ref=agbuf_ref.at[pl.ds(s * CHUNK, CHUNK), :],
                send_sem=ag_send.at[0],
                recv_sem=ag_recv.at[s],
                device_id=(my,),
                device_id_type=pl.DeviceIdType.MESH,
            )
            recv.wait_recv()

        for desc in rs_descs + ag_descs:
            desc.wait_send()

        for b in range(B):
            out_ref[b] = agbuf_ref[pl.ds(b * SQ, SQ), :].astype(jnp.float32)

    return pl.pallas_call(
        body,
        out_shape=jax.ShapeDtypeStruct((B, SQ, D_MODEL), jnp.float32),
        in_specs=[pl.BlockSpec(memory_space=pltpu.VMEM)] * 5,
        out_specs=pl.BlockSpec(memory_space=pltpu.VMEM),
        scratch_shapes=[
            pltpu.VMEM((ROWS, D_MODEL), jnp.float32),
            pltpu.VMEM((ROWS, D_MODEL), jnp.bfloat16),
            pltpu.VMEM((N_DEV, CHUNK, D_MODEL), jnp.bfloat16),
            pltpu.VMEM((ROWS, D_MODEL), jnp.bfloat16),
            pltpu.SemaphoreType.DMA((N_DEV,)),
            pltpu.SemaphoreType.DMA((N_DEV,)),
            pltpu.SemaphoreType.DMA((N_DEV,)),
            pltpu.SemaphoreType.DMA((N_DEV,)),
        ],
        compiler_params=pltpu.CompilerParams(collective_id=0),
    )(xb, wq, K_loc, V_loc, wo)


# device time: 26208 ns/iter; 3.6675x vs baseline; 1.3790x over previous
import jax
import jax.numpy as jnp
from jax import lax
from jax.experimental import pallas as pl
from jax.experimental.pallas import tpu as pltpu

N_DEV = 16
B, SQ, D_MODEL = 2, 256, 512
HL, DH = 4, 64
BLK = 64
NBLK = SQ // BLK
ROWS = B * SQ
CHUNK = ROWS // N_DEV


def kernel(x, Wq, K_ext, V_ext, Wo):
    p = lax.axis_index("i")
    K_loc = lax.dynamic_slice_in_dim(K_ext, p * HL, HL, axis=2).astype(
        jnp.bfloat16)
    V_loc = lax.dynamic_slice_in_dim(V_ext, p * HL, HL, axis=2).astype(
        jnp.bfloat16)

    def body(x_ref, wq_ref, k_ref, v_ref, wo_ref, out_ref,
             acc_ref, sbuf_ref, comm_ref, agbuf_ref, ctx_ref,
             rs_send, rs_recv, ag_send, ag_recv):
        my = lax.axis_index("i")

        barrier_sem = pltpu.get_barrier_semaphore()
        for d in range(1, N_DEV):
            q = lax.rem(my + d, N_DEV)
            pl.semaphore_signal(barrier_sem, inc=1, device_id=(q,),
                                device_id_type=pl.DeviceIdType.MESH)

        wq_b = (wq_ref[...] * 0.125).astype(jnp.bfloat16)
        wo_b = wo_ref[...].astype(jnp.bfloat16)
        x2 = x_ref[...].reshape(ROWS, D_MODEL).astype(jnp.bfloat16)
        q_all = lax.dot(x2, wq_b,
                        preferred_element_type=jnp.float32)
        q_all_b = q_all.astype(jnp.bfloat16)
        qbi = lax.broadcasted_iota(jnp.int32, (SQ, SQ), 0) // BLK
        kbi = lax.broadcasted_iota(jnp.int32, (SQ, SQ), 1) // BLK
        negmask = jnp.where(qbi == kbi, 0.0, -1e9).astype(jnp.float32)
        for b in range(B):
            kh_all = k_ref[b].reshape(SQ, HL * DH)
            vh_all = v_ref[b].reshape(SQ, HL * DH)
            for h in range(HL):
                qh = q_all_b[b * SQ:(b + 1) * SQ, h * DH:(h + 1) * DH]
                kh = kh_all[:, h * DH:(h + 1) * DH]
                s = lax.dot_general(
                    qh, kh, (((1,), (1,)), ((), ())),
                    preferred_element_type=jnp.float32)
                e = jnp.exp(s + negmask)
                w = e / jnp.sum(e, axis=1, keepdims=True)
                ctx_ref[b * SQ:(b + 1) * SQ, h * DH:(h + 1) * DH] = lax.dot(
                    w.astype(jnp.bfloat16), vh_all[:, h * DH:(h + 1) * DH],
                    preferred_element_type=jnp.float32)

        part = lax.dot(ctx_ref[...].astype(jnp.bfloat16), wo_b,
                       preferred_element_type=jnp.float32)
        acc_ref[...] = part
        sbuf_ref[...] = part.astype(jnp.bfloat16)

        pl.semaphore_wait(barrier_sem, N_DEV - 1)

        rs_descs = []
        for d in range(1, N_DEV):
            q = lax.rem(my + d, N_DEV)
            desc = pltpu.make_async_remote_copy(
                src_ref=sbuf_ref.at[pl.ds(q * CHUNK, CHUNK), :],
                dst_ref=comm_ref.at[my],
                send_sem=rs_send.at[d],
                recv_sem=rs_recv.at[my],
                device_id=(q,),
                device_id_type=pl.DeviceIdType.MESH,
            )
            desc.start()
            rs_descs.append(desc)

        comm_ref[my] = jnp.zeros((CHUNK, D_MODEL), jnp.bfloat16)
        for d in range(1, N_DEV):
            s = lax.rem(my + d, N_DEV)
            recv = pltpu.make_async_remote_copy(
                src_ref=sbuf_ref.at[pl.ds(0, CHUNK), :],
                dst_ref=comm_ref.at[s],
                send_sem=rs_send.at[0],
                recv_sem=rs_recv.at[s],
                device_id=(my,),
                device_id_type=pl.DeviceIdType.MESH,
            )
            recv.wait_recv()

        total = (acc_ref[pl.ds(my * CHUNK, CHUNK), :]
                 + jnp.sum(comm_ref[...].astype(jnp.float32), axis=0))
        agbuf_ref[pl.ds(my * CHUNK, CHUNK), :] = total.astype(jnp.bfloat16)

        ag_descs = []
        for d in range(1, N_DEV):
            q = lax.rem(my + d, N_DEV)
            desc = pltpu.make_async_remote_copy(
                src_ref=agbuf_ref.at[pl.ds(my * CHUNK, CHUNK), :],
                dst_ref=agbuf_ref.at[pl.ds(my * CHUNK, CHUNK), :],
                send_sem=ag_send.at[d],
                recv_sem=ag_recv.at[my],
                device_id=(q,),
                device_id_type=pl.DeviceIdType.MESH,
            )
            desc.start()
            ag_descs.append(desc)

        for d in range(1, N_DEV):
            s = lax.rem(my + d, N_DEV)
            recv = pltpu.make_async_remote_copy(
                src_ref=sbuf_ref.at[pl.ds(0, CHUNK), :],
                dst_ref=agbuf_ref.at[pl.ds(s * CHUNK, CHUNK), :],
                send_sem=ag_send.at[0],
                recv_sem=ag_recv.at[s],
                device_id=(my,),
                device_id_type=pl.DeviceIdType.MESH,
            )
            recv.wait_recv()

        for desc in rs_descs + ag_descs:
            desc.wait_send()

        for b in range(B):
            out_ref[b] = agbuf_ref[pl.ds(b * SQ, SQ), :].astype(jnp.float32)

    return pl.pallas_call(
        body,
        out_shape=jax.ShapeDtypeStruct((B, SQ, D_MODEL), jnp.float32),
        in_specs=[pl.BlockSpec(memory_space=pltpu.VMEM)] * 5,
        out_specs=pl.BlockSpec(memory_space=pltpu.VMEM),
        scratch_shapes=[
            pltpu.VMEM((ROWS, D_MODEL), jnp.float32),
            pltpu.VMEM((ROWS, D_MODEL), jnp.bfloat16),
            pltpu.VMEM((N_DEV, CHUNK, D_MODEL), jnp.bfloat16),
            pltpu.VMEM((ROWS, D_MODEL), jnp.bfloat16),
            pltpu.VMEM((ROWS, HL * DH), jnp.float32),
            pltpu.SemaphoreType.DMA((N_DEV,)),
            pltpu.SemaphoreType.DMA((N_DEV,)),
            pltpu.SemaphoreType.DMA((N_DEV,)),
            pltpu.SemaphoreType.DMA((N_DEV,)),
        ],
        compiler_params=pltpu.CompilerParams(collective_id=0),
    )(x, Wq, K_loc, V_loc, Wo)
